# initial kernel scaffold (unmeasured)
import functools

import jax
import jax.numpy as jnp
from jax import lax
from jax.experimental import pallas as pl
from jax.experimental.pallas import tpu as pltpu

N_DEV = 4


def kernel(A, B):
    m, k_shard = A.shape
    _, n = B.shape
    m_chunk = m // N_DEV

    def body(a_ref, b_ref, out_ref, b_bf_ref, comm_ref, send_sems, recv_sems):
        my = lax.axis_index("i")
        left = (my + N_DEV - 1) % N_DEV
        right = (my + 1) % N_DEV

        barrier_sem = pltpu.get_barrier_semaphore()
        for nbr in (left, right):
            pl.semaphore_signal(
                barrier_sem, inc=1,
                device_id=(nbr,), device_id_type=pl.DeviceIdType.MESH,
            )
        pl.semaphore_wait(barrier_sem, 2)

        b_bf_ref[...] = b_ref[...].astype(jnp.bfloat16)

        def partial_chunk(c):
            a = a_ref[pl.ds(c * m_chunk, m_chunk), :].astype(jnp.bfloat16)
            return jnp.dot(a, b_bf_ref[...], preferred_element_type=jnp.float32)

        comm_ref[N_DEV - 1] = partial_chunk(
            (my + N_DEV - 1) % N_DEV
        ).astype(jnp.bfloat16)

        for t in range(N_DEV - 1):
            src_slot = (N_DEV - 1) if t == 0 else t - 1
            rdma = pltpu.make_async_remote_copy(
                src_ref=comm_ref.at[src_slot],
                dst_ref=comm_ref.at[t],
                send_sem=send_sems.at[t],
                recv_sem=recv_sems.at[t],
                device_id=(right,),
                device_id_type=pl.DeviceIdType.MESH,
            )
            rdma.start()
            p = partial_chunk((my + 2 - t) % N_DEV)
            rdma.wait()
            if t < N_DEV - 2:
                comm_ref[t] = (
                    comm_ref[t].astype(jnp.float32) + p
                ).astype(jnp.bfloat16)
            else:
                out_ref[...] = comm_ref[t].astype(jnp.float32) + p

        @functools.partial(pl.run_scoped, sem=pltpu.SemaphoreType.REGULAR)
        def _(sem):
            for nbr in (left, right):
                pl.semaphore_signal(
                    sem, inc=1,
                    device_id=(nbr,), device_id_type=pl.DeviceIdType.MESH,
                )
            pl.semaphore_wait(sem, 2)

    return pl.pallas_call(
        body,
        out_shape=jax.ShapeDtypeStruct((m_chunk, n), jnp.float32),
        in_specs=[
            pl.BlockSpec(memory_space=pltpu.VMEM),
            pl.BlockSpec(memory_space=pltpu.VMEM),
        ],
        out_specs=pl.BlockSpec(memory_space=pltpu.VMEM),
        scratch_shapes=[
            pltpu.VMEM((k_shard, n), jnp.bfloat16),
            pltpu.VMEM((N_DEV, m_chunk, n), jnp.bfloat16),
            pltpu.SemaphoreType.DMA((N_DEV - 1,)),
            pltpu.SemaphoreType.DMA((N_DEV - 1,)),
        ],
        compiler_params=pltpu.CompilerParams(collective_id=0),
    )(A, B)


# baseline (device time: 190385 ns/iter reference)
import functools

import jax
import jax.numpy as jnp
from jax import lax
from jax.experimental import pallas as pl
from jax.experimental.pallas import tpu as pltpu

N_DEV = 4
N_SUB = 4


def kernel(A, B):
    m, k_shard = A.shape
    _, n = B.shape
    m_chunk = m // N_DEV
    n_sub = n // N_SUB
    k_piece = k_shard // N_SUB

    def body(a_hbm, b_hbm, out_ref,
             a_stage, a_bf, b_stage, b_bf, acc, comm_ref,
             local_sem, send_sems, recv_sems):
        my = lax.axis_index("i")
        left = (my + N_DEV - 1) % N_DEV
        right = (my + 1) % N_DEV

        barrier_sem = pltpu.get_barrier_semaphore()
        for nbr in (left, right):
            pl.semaphore_signal(
                barrier_sem, inc=1,
                device_id=(nbr,), device_id_type=pl.DeviceIdType.MESH,
            )
        pl.semaphore_wait(barrier_sem, 2)

        for jp in range(N_SUB):
            ks = pl.ds(jp * k_piece, k_piece)
            cp = pltpu.make_async_copy(b_hbm.at[ks, :], b_stage, local_sem)
            cp.start()
            cp.wait()
            b_bf[ks, :] = b_stage[...].astype(jnp.bfloat16)

        def load_a_chunk(c):
            cp = pltpu.make_async_copy(
                a_hbm.at[pl.ds(c * m_chunk, m_chunk), :], a_stage, local_sem
            )
            cp.start()
            cp.wait()
            a_bf[...] = a_stage[...].astype(jnp.bfloat16)

        def compute_partial():
            for j in range(N_SUB):
                js = pl.ds(j * n_sub, n_sub)
                acc[:, js] = jnp.dot(
                    a_bf[...], b_bf[:, js], preferred_element_type=jnp.float32
                )

        load_a_chunk((my + N_DEV - 1) % N_DEV)
        for j in range(N_SUB):
            js = pl.ds(j * n_sub, n_sub)
            out_ref[:, js] = jnp.dot(
                a_bf[...], b_bf[:, js], preferred_element_type=jnp.float32
            ).astype(jnp.bfloat16)

        for t in range(N_DEV - 1):
            src = out_ref if t == 0 else comm_ref.at[t - 1]
            rdma = pltpu.make_async_remote_copy(
                src_ref=src,
                dst_ref=comm_ref.at[t],
                send_sem=send_sems.at[t],
                recv_sem=recv_sems.at[t],
                device_id=(right,),
                device_id_type=pl.DeviceIdType.MESH,
            )
            rdma.start()
            load_a_chunk((my + 2 - t) % N_DEV)
            compute_partial()
            rdma.wait()
            for j in range(N_SUB):
                js = pl.ds(j * n_sub, n_sub)
                s = comm_ref[t, :, js].astype(jnp.float32) + acc[:, js]
                if t < N_DEV - 2:
                    comm_ref[t, :, js] = s.astype(jnp.bfloat16)
                else:
                    out_ref[:, js] = s.astype(jnp.bfloat16)

        @functools.partial(pl.run_scoped, sem=pltpu.SemaphoreType.REGULAR)
        def _(sem):
            for nbr in (left, right):
                pl.semaphore_signal(
                    sem, inc=1,
                    device_id=(nbr,), device_id_type=pl.DeviceIdType.MESH,
                )
            pl.semaphore_wait(sem, 2)

    return pl.pallas_call(
        body,
        out_shape=jax.ShapeDtypeStruct((m_chunk, n), jnp.bfloat16),
        in_specs=[
            pl.BlockSpec(memory_space=pl.ANY),
            pl.BlockSpec(memory_space=pl.ANY),
        ],
        out_specs=pl.BlockSpec(memory_space=pltpu.VMEM),
        scratch_shapes=[
            pltpu.VMEM((m_chunk, k_shard), jnp.float32),
            pltpu.VMEM((m_chunk, k_shard), jnp.bfloat16),
            pltpu.VMEM((k_piece, n), jnp.float32),
            pltpu.VMEM((k_shard, n), jnp.bfloat16),
            pltpu.VMEM((m_chunk, n), jnp.float32),
            pltpu.VMEM((N_DEV - 1, m_chunk, n), jnp.bfloat16),
            pltpu.SemaphoreType.DMA,
            pltpu.SemaphoreType.DMA((N_DEV - 1,)),
            pltpu.SemaphoreType.DMA((N_DEV - 1,)),
        ],
        compiler_params=pltpu.CompilerParams(
            collective_id=0,
            vmem_limit_bytes=56 * 1024 * 1024,
        ),
    )(A, B)


# device time: 129703 ns/iter; 1.4679x vs baseline; 1.4679x over previous
import functools

import jax
import jax.numpy as jnp
from jax import lax
from jax.experimental import pallas as pl
from jax.experimental.pallas import tpu as pltpu

N_DEV = 4
N_SUB = 768


def kernel(A, B):
    m, k_shard = A.shape
    _, n = B.shape
    m_chunk = m // N_DEV
    half = n // 2

    def body(a_ref, b_ref, out_ref, acc, comm_cw, comm_ccw,
             send_cw, recv_cw, send_ccw, recv_ccw):
        my = lax.axis_index("i")
        left = (my + N_DEV - 1) % N_DEV
        right = (my + 1) % N_DEV

        barrier_sem = pltpu.get_barrier_semaphore()
        for nbr in (left, right):
            pl.semaphore_signal(
                barrier_sem, inc=1,
                device_id=(nbr,), device_id_type=pl.DeviceIdType.MESH,
            )
        pl.semaphore_wait(barrier_sem, 2)

        def partial_cols(c, col0, col1, dst_ref, as_bf16):
            a = a_ref[pl.ds(c * m_chunk, m_chunk), :]
            for j in range(col0, col1, N_SUB):
                js = pl.ds(j, N_SUB)
                v = jnp.dot(a, b_ref[:, js], preferred_element_type=jnp.float32)
                dst_ref[:, js] = v.astype(jnp.bfloat16) if as_bf16 else v

        partial_cols((my + N_DEV - 1) % N_DEV, 0, half, out_ref, True)
        partial_cols((my + 1) % N_DEV, half, n, out_ref, True)

        for t in range(N_DEV - 1):
            src_cw = (
                out_ref.at[:, pl.ds(0, half)] if t == 0 else comm_cw.at[t - 1]
            )
            src_ccw = (
                out_ref.at[:, pl.ds(half, half)] if t == 0 else comm_ccw.at[t - 1]
            )
            rdma_cw = pltpu.make_async_remote_copy(
                src_ref=src_cw, dst_ref=comm_cw.at[t],
                send_sem=send_cw.at[t], recv_sem=recv_cw.at[t],
                device_id=(right,), device_id_type=pl.DeviceIdType.MESH,
            )
            rdma_ccw = pltpu.make_async_remote_copy(
                src_ref=src_ccw, dst_ref=comm_ccw.at[t],
                send_sem=send_ccw.at[t], recv_sem=recv_ccw.at[t],
                device_id=(left,), device_id_type=pl.DeviceIdType.MESH,
            )
            rdma_cw.start()
            rdma_ccw.start()

            if t == 1:
                partial_cols((my + 1) % N_DEV, 0, half, acc, False)
                partial_cols((my + N_DEV - 1) % N_DEV, half, n, acc, False)
            else:
                partial_cols((my + 2 - t) % N_DEV, 0, n, acc, False)

            rdma_cw.wait()
            rdma_ccw.wait()

            last = t == N_DEV - 2
            for j in range(0, half, N_SUB):
                js = pl.ds(j, N_SUB)
                js_r = pl.ds(half + j, N_SUB)
                s_cw = comm_cw[t, :, js].astype(jnp.float32) + acc[:, js]
                s_ccw = comm_ccw[t, :, js].astype(jnp.float32) + acc[:, js_r]
                if last:
                    out_ref[:, js] = s_cw.astype(jnp.bfloat16)
                    out_ref[:, js_r] = s_ccw.astype(jnp.bfloat16)
                else:
                    comm_cw[t, :, js] = s_cw.astype(jnp.bfloat16)
                    comm_ccw[t, :, js] = s_ccw.astype(jnp.bfloat16)

        @functools.partial(pl.run_scoped, sem=pltpu.SemaphoreType.REGULAR)
        def _(sem):
            for nbr in (left, right):
                pl.semaphore_signal(
                    sem, inc=1,
                    device_id=(nbr,), device_id_type=pl.DeviceIdType.MESH,
                )
            pl.semaphore_wait(sem, 2)

    call = pl.pallas_call(
        body,
        out_shape=jax.ShapeDtypeStruct((m_chunk, n), jnp.bfloat16),
        in_specs=[
            pl.BlockSpec(memory_space=pltpu.MemorySpace.VMEM),
            pl.BlockSpec(memory_space=pltpu.MemorySpace.VMEM),
        ],
        out_specs=pl.BlockSpec(memory_space=pltpu.MemorySpace.VMEM),
        scratch_shapes=[
            pltpu.VMEM((m_chunk, n), jnp.float32),
            pltpu.VMEM((N_DEV - 1, m_chunk, half), jnp.bfloat16),
            pltpu.VMEM((N_DEV - 1, m_chunk, half), jnp.bfloat16),
            pltpu.SemaphoreType.DMA((N_DEV - 1,)),
            pltpu.SemaphoreType.DMA((N_DEV - 1,)),
            pltpu.SemaphoreType.DMA((N_DEV - 1,)),
            pltpu.SemaphoreType.DMA((N_DEV - 1,)),
        ],
        compiler_params=pltpu.CompilerParams(
            collective_id=0,
            vmem_limit_bytes=56 * 1024 * 1024,
        ),
    )
    return call(A.astype(jnp.bfloat16), B.astype(jnp.bfloat16))
